# baseline (speedup 1.0000x reference)
import functools

import jax
import jax.numpy as jnp
from jax import lax
from jax.experimental import pallas as pl
from jax.experimental.pallas import tpu as pltpu
from jax.experimental.pallas import tpu_sc as plsc

N = 10000
E = 320000
D = 128
DA = 160
NS = 16
E_TILE = E // NS
CHUNK = 128
BCH = 4
NB = 40
NCH = NB * BCH
E_PAD = NCH * CHUNK
ACC_ROWS = 10240
ROWS_PER_TILE = ACC_ROWS // NS


def _agg_body(table, srcs, dsts, out, ixs0, ixs1, ixd0, ixd1, rows0, rows1,
              tb, acc, isem0, isem1, gsem0, gsem1, ssem0, ssem1):
    c = lax.axis_index("c")
    s = lax.axis_index("s")
    ixs = (ixs0, ixs1)
    ixd = (ixd0, ixd1)
    rows = (rows0, rows1)
    isem = (isem0, isem1)
    gsem = (gsem0, gsem1)
    ssem = (ssem0, ssem1)
    base = s * ROWS_PER_TILE

    def stage_block(off, nr):
        pltpu.sync_copy(table.at[c, pl.ds(off, nr)], rows0.at[pl.ds(0, nr)])
        pltpu.sync_copy(rows0.at[pl.ds(0, nr)], tb.at[pl.ds(off, nr)])

    @pl.when(s < NS - 1)
    def _stage_full():
        for k in range(ROWS_PER_TILE // CHUNK):
            stage_block(base + k * CHUNK, CHUNK)

    @pl.when(s == NS - 1)
    def _stage_last():
        tail = N - (NS - 1) * ROWS_PER_TILE
        for k in range(tail // CHUNK):
            stage_block(base + k * CHUNK, CHUNK)
        rem = tail - (tail // CHUNK) * CHUNK
        if rem:
            stage_block(base + (tail // CHUNK) * CHUNK, rem)

    zero32 = jnp.zeros((32,), jnp.bfloat16)

    def zrow(i, _):
        def zcol(j, _):
            rows1[i, pl.ds(j * 32, 32)] = zero32
            return 0
        return lax.fori_loop(0, DA // 32, zcol, 0)

    lax.fori_loop(0, CHUNK, zrow, 0)
    nfull = ROWS_PER_TILE // CHUNK
    for k in range(nfull):
        pltpu.sync_copy(rows1, acc.at[pl.ds(base + k * CHUNK, CHUNK)])
    rem = ROWS_PER_TILE - nfull * CHUNK
    if rem:
        pltpu.sync_copy(rows1.at[pl.ds(0, rem)],
                        acc.at[pl.ds(base + nfull * CHUNK, rem)])

    plsc.subcore_barrier()

    def load_batch(m, a):
        pltpu.async_copy(srcs.at[c, s, m], ixs[a], isem[a])
        pltpu.async_copy(dsts.at[c, s, m], ixd[a], isem[a])

    def wait_batch(a):
        pltpu.make_async_copy(srcs.at[c, s, 0], ixs[a], isem[a]).wait()
        pltpu.make_async_copy(dsts.at[c, s, 0], ixd[a], isem[a]).wait()

    def gather(a, k, b):
        pltpu.async_copy(tb.at[ixs[a].at[k]], rows[b], gsem[b])

    def wait_gather(a, k, b):
        pltpu.make_async_copy(tb.at[ixs[a].at[k]], rows[b], gsem[b]).wait()

    load_batch(0, 0)
    load_batch(1, 1)
    wait_batch(0)
    gather(0, 0, 0)
    gather(0, 1, 1)

    def superstep(t, _):
        for a in (0, 1):
            m = 2 * t + a
            for k in range(BCH):
                b = k % 2
                wait_gather(a, k, b)
                pltpu.async_copy(rows[b], acc.at[ixd[a].at[k]], ssem[b],
                                 add=True)
                pltpu.make_async_copy(rows[b], acc.at[ixd[a].at[k]],
                                      ssem[b]).wait()
                if k < BCH - 2:
                    gather(a, k + 2, b)
                else:
                    if k == BCH - 2:
                        @pl.when(m < NB - 1)
                        def _wait_next_idx():
                            wait_batch(1 - a)

                    @pl.when(m < NB - 1)
                    def _cross_batch_gather():
                        gather(1 - a, k + 2 - BCH, b)

            @pl.when(m + 2 < NB)
            def _prefetch_idx():
                load_batch(m + 2, a)
        return 0

    lax.fori_loop(0, NB // 2, superstep, 0)

    plsc.subcore_barrier()

    @pl.when(s < NS - 1)
    def _full_tile():
        pltpu.sync_copy(acc.at[pl.ds(base, ROWS_PER_TILE)],
                        out.at[c, pl.ds(base, ROWS_PER_TILE)])

    @pl.when(s == NS - 1)
    def _last_tile():
        tail = N - (NS - 1) * ROWS_PER_TILE
        pltpu.sync_copy(acc.at[pl.ds(base, tail)],
                        out.at[c, pl.ds(base, tail)])


def _make_agg():
    mesh = plsc.VectorSubcoreMesh(core_axis_name="c", subcore_axis_name="s",
                                  num_cores=2, num_subcores=NS)
    return pl.kernel(
        _agg_body,
        out_type=jax.ShapeDtypeStruct((2, N, DA), jnp.bfloat16),
        mesh=mesh,
        scratch_types=[
            pltpu.VMEM((BCH, CHUNK), jnp.int32),
            pltpu.VMEM((BCH, CHUNK), jnp.int32),
            pltpu.VMEM((BCH, CHUNK), jnp.int32),
            pltpu.VMEM((BCH, CHUNK), jnp.int32),
            pltpu.VMEM((CHUNK, DA), jnp.bfloat16),
            pltpu.VMEM((CHUNK, DA), jnp.bfloat16),
            pltpu.VMEM_SHARED((ACC_ROWS, DA), jnp.bfloat16),
            pltpu.VMEM_SHARED((ACC_ROWS, DA), jnp.bfloat16),
            pltpu.SemaphoreType.DMA,
            pltpu.SemaphoreType.DMA,
            pltpu.SemaphoreType.DMA,
            pltpu.SemaphoreType.DMA,
            pltpu.SemaphoreType.DMA,
            pltpu.SemaphoreType.DMA,
        ],
        compiler_params=pltpu.CompilerParams(use_tc_tiling_on_sc=False),
    )


R_BLK = 2000


def _proj_body(x_ref, w_ref, b_ref, o_ref):
    y = jnp.maximum(
        jnp.dot(x_ref[0], w_ref[0], preferred_element_type=jnp.float32)
        + b_ref[0], 0.0)
    o_ref[0, :, :D] = y.astype(jnp.bfloat16)
    o_ref[0, :, D:DA] = jnp.ones((y.shape[0], DA - D), jnp.bfloat16)


def _proj(xs, wp, bp):
    return pl.pallas_call(
        _proj_body,
        grid=(2, N // R_BLK),
        in_specs=[
            pl.BlockSpec((1, R_BLK, D), lambda t, r: (t, r, 0)),
            pl.BlockSpec((1, D, D), lambda t, r: (t, 0, 0)),
            pl.BlockSpec((1, 1, D), lambda t, r: (t, 0, 0)),
        ],
        out_specs=pl.BlockSpec((1, R_BLK, DA), lambda t, r: (t, r, 0)),
        out_shape=jax.ShapeDtypeStruct((2, N, DA), jnp.bfloat16),
    )(xs, wp, bp)


def _sage_block(agg_ref, h_ref, wl_ref, wr_ref, b_ref):
    x = agg_ref[0].astype(jnp.float32)
    inv = 1.0 / jnp.maximum(x[:, D:D + 1], 1.0)
    mean = x[:, :D] * inv
    h = h_ref[0][:, :D].astype(jnp.float32)
    z = (jnp.dot(mean, wl_ref[0], preferred_element_type=jnp.float32)
         + jnp.dot(h, wr_ref[0], preferred_element_type=jnp.float32)
         + b_ref[0])
    return jnp.maximum(z, 0.0)


def _store_aug(o_ref, y):
    o_ref[0, :, :D] = y.astype(jnp.bfloat16)
    o_ref[0, :, D:DA] = jnp.ones((y.shape[0], DA - D), jnp.bfloat16)


def _combine_body(agg_ref, h_ref, wl_ref, wr_ref, b_ref, o_ref):
    _store_aug(o_ref, _sage_block(agg_ref, h_ref, wl_ref, wr_ref, b_ref))


_SAGE_SPECS = [
    pl.BlockSpec((1, R_BLK, DA), lambda t, r: (1 - t, r, 0)),
    pl.BlockSpec((1, R_BLK, DA), lambda t, r: (t, r, 0)),
    pl.BlockSpec((1, D, D), lambda t, r: (t, 0, 0)),
    pl.BlockSpec((1, D, D), lambda t, r: (t, 0, 0)),
    pl.BlockSpec((1, 1, D), lambda t, r: (t, 0, 0)),
]


def _combine(agg, h, wl, wr, b):
    return pl.pallas_call(
        _combine_body,
        grid=(2, N // R_BLK),
        in_specs=_SAGE_SPECS,
        out_specs=pl.BlockSpec((1, R_BLK, DA), lambda t, r: (t, r, 0)),
        out_shape=jax.ShapeDtypeStruct((2, N, DA), jnp.bfloat16),
    )(agg, h, wl, wr, b)


def _combine_final_body(agg_ref, h_ref, wl_ref, wr_ref, b_ref, w1_ref,
                        b1_ref, w2_ref, b2_ref, out_ref, cs_ref):
    t = pl.program_id(0)
    r = pl.program_id(1)
    y = _sage_block(agg_ref, h_ref, wl_ref, wr_ref, b_ref)

    @pl.when((t == 0) & (r == 0))
    def _init():
        cs_ref[...] = jnp.zeros_like(cs_ref)

    cs_ref[...] += jnp.sum(y, axis=0, keepdims=True)

    @pl.when((t == 1) & (r == pl.num_programs(1) - 1))
    def _readout():
        ge = cs_ref[...] * (0.5 / N)
        z = jnp.maximum(
            jnp.dot(ge, w1_ref[...], preferred_element_type=jnp.float32)
            + b1_ref[...], 0.0)
        out_ref[...] = (jnp.dot(z, w2_ref[...],
                                preferred_element_type=jnp.float32)
                        + b2_ref[...])


def _combine_final(agg, h, wl, wr, b, w1, b1, w2, b2):
    C = w2.shape[1]
    return pl.pallas_call(
        _combine_final_body,
        grid=(2, N // R_BLK),
        in_specs=_SAGE_SPECS + [
            pl.BlockSpec(w1.shape, lambda t, r: (0, 0)),
            pl.BlockSpec(b1.shape, lambda t, r: (0, 0)),
            pl.BlockSpec(w2.shape, lambda t, r: (0, 0)),
            pl.BlockSpec(b2.shape, lambda t, r: (0, 0)),
        ],
        out_specs=pl.BlockSpec((1, C), lambda t, r: (0, 0)),
        out_shape=jax.ShapeDtypeStruct((1, C), jnp.float32),
        scratch_shapes=[pltpu.VMEM((1, D), jnp.float32)],
    )(agg, h, wl, wr, b, w1, b1, w2, b2)


def _pad_idx(idx, fill, offset):
    a = idx.astype(jnp.int32).reshape(NS, E_TILE) + offset
    pad = jnp.full((NS, E_PAD - E_TILE), fill, jnp.int32)
    return jnp.concatenate([a, pad], axis=1).reshape(NS, NB, BCH, CHUNK)


def kernel(x_user, x_item, edge_index_ui, edge_index_iu, params):
    p = params

    xs = jnp.stack([x_user, x_item])
    wp = jnp.stack([p["Wp_user"], p["Wp_item"]])
    bp = jnp.stack([p["bp_user"], p["bp_item"]])[:, None, :]

    inv_eps = 1.0 / jnp.sqrt(jnp.float32(1.0 + 1e-5))
    wl, wr, bb = [], [], []
    for l in range(2):
        s_u = p[f"g_user_{l}"] * inv_eps
        s_i = p[f"g_item_{l}"] * inv_eps
        wl.append(jnp.stack([p[f"Wl_iu_{l}"] * s_u, p[f"Wl_ui_{l}"] * s_i]))
        wr.append(jnp.stack([p[f"Wr_iu_{l}"] * s_u, p[f"Wr_ui_{l}"] * s_i]))
        bb.append(jnp.stack([
            p[f"bl_iu_{l}"] * s_u + p[f"be_user_{l}"],
            p[f"bl_ui_{l}"] * s_i + p[f"be_item_{l}"],
        ])[:, None, :])

    src_idx = jnp.stack([
        _pad_idx(edge_index_ui[0], 0, 0),
        _pad_idx(edge_index_iu[0], 0, 0),
    ])
    dst_idx = jnp.stack([
        _pad_idx(edge_index_ui[1], N, 0),
        _pad_idx(edge_index_iu[1], N, 0),
    ])

    agg_fn = _make_agg()

    h = _proj(xs, wp, bp)
    agg = agg_fn(h, src_idx, dst_idx)
    h = _combine(agg, h, wl[0], wr[0], bb[0])
    agg = agg_fn(h, src_idx, dst_idx)
    out = _combine_final(agg, h, wl[1], wr[1], bb[1],
                         p["W1"], p["b1"][None, :], p["W2"], p["b2"][None, :])
    return (out, jnp.zeros((), jnp.float32))


# speedup vs baseline: 1.0238x; 1.0238x over previous
import functools

import jax
import jax.numpy as jnp
from jax import lax
from jax.experimental import pallas as pl
from jax.experimental.pallas import tpu as pltpu
from jax.experimental.pallas import tpu_sc as plsc

N = 10000
E = 320000
D = 128
DA = 160
NS = 16
E_TILE = E // NS
CHUNK = 128
BCH = 8
NB = 20
NCH = NB * BCH
E_PAD = NCH * CHUNK
ACC_ROWS = 10240
ROWS_PER_TILE = ACC_ROWS // NS


def _agg_body(table, srcs, dsts, out, ixs0, ixs1, ixd0, ixd1, rows0, rows1,
              tb, acc, isem0, isem1, gsem0, gsem1, ssem0, ssem1):
    c = lax.axis_index("c")
    s = lax.axis_index("s")
    ixs = (ixs0, ixs1)
    ixd = (ixd0, ixd1)
    rows = (rows0, rows1)
    isem = (isem0, isem1)
    gsem = (gsem0, gsem1)
    ssem = (ssem0, ssem1)
    base = s * ROWS_PER_TILE

    def stage_block(off, nr):
        pltpu.sync_copy(table.at[c, pl.ds(off, nr)], tb.at[pl.ds(off, nr)])

    @pl.when(s < NS - 1)
    def _stage_full():
        stage_block(base, ROWS_PER_TILE)

    @pl.when(s == NS - 1)
    def _stage_last():
        stage_block(base, N - (NS - 1) * ROWS_PER_TILE)

    zero32 = jnp.zeros((32,), jnp.bfloat16)

    def zrow(i, _):
        def zcol(j, _):
            rows1[i, pl.ds(j * 32, 32)] = zero32
            return 0
        return lax.fori_loop(0, DA // 32, zcol, 0)

    lax.fori_loop(0, CHUNK, zrow, 0)
    nfull = ROWS_PER_TILE // CHUNK
    for k in range(nfull):
        pltpu.sync_copy(rows1, acc.at[pl.ds(base + k * CHUNK, CHUNK)])
    rem = ROWS_PER_TILE - nfull * CHUNK
    if rem:
        pltpu.sync_copy(rows1.at[pl.ds(0, rem)],
                        acc.at[pl.ds(base + nfull * CHUNK, rem)])

    plsc.subcore_barrier()

    def load_batch(m, a):
        pltpu.async_copy(srcs.at[c, s, m], ixs[a], isem[a])
        pltpu.async_copy(dsts.at[c, s, m], ixd[a], isem[a])

    def wait_batch(a):
        pltpu.make_async_copy(srcs.at[c, s, 0], ixs[a], isem[a]).wait()
        pltpu.make_async_copy(dsts.at[c, s, 0], ixd[a], isem[a]).wait()

    def gather(a, k, b):
        pltpu.async_copy(tb.at[ixs[a].at[k]], rows[b], gsem[b])

    def wait_gather(a, k, b):
        pltpu.make_async_copy(tb.at[ixs[a].at[k]], rows[b], gsem[b]).wait()

    load_batch(0, 0)
    load_batch(1, 1)
    wait_batch(0)
    gather(0, 0, 0)
    gather(0, 1, 1)

    def superstep(t, _):
        for a in (0, 1):
            m = 2 * t + a
            for k in range(BCH):
                b = k % 2
                wait_gather(a, k, b)
                pltpu.async_copy(rows[b], acc.at[ixd[a].at[k]], ssem[b],
                                 add=True)
                pltpu.make_async_copy(rows[b], acc.at[ixd[a].at[k]],
                                      ssem[b]).wait()
                if k < BCH - 2:
                    gather(a, k + 2, b)
                else:
                    if k == BCH - 2:
                        @pl.when(m < NB - 1)
                        def _wait_next_idx():
                            wait_batch(1 - a)

                    @pl.when(m < NB - 1)
                    def _cross_batch_gather():
                        gather(1 - a, k + 2 - BCH, b)

            @pl.when(m + 2 < NB)
            def _prefetch_idx():
                load_batch(m + 2, a)
        return 0

    lax.fori_loop(0, NB // 2, superstep, 0)

    plsc.subcore_barrier()

    @pl.when(s < NS - 1)
    def _full_tile():
        pltpu.sync_copy(acc.at[pl.ds(base, ROWS_PER_TILE)],
                        out.at[c, pl.ds(base, ROWS_PER_TILE)])

    @pl.when(s == NS - 1)
    def _last_tile():
        tail = N - (NS - 1) * ROWS_PER_TILE
        pltpu.sync_copy(acc.at[pl.ds(base, tail)],
                        out.at[c, pl.ds(base, tail)])


def _make_agg():
    mesh = plsc.VectorSubcoreMesh(core_axis_name="c", subcore_axis_name="s",
                                  num_cores=2, num_subcores=NS)
    return pl.kernel(
        _agg_body,
        out_type=jax.ShapeDtypeStruct((2, N, DA), jnp.bfloat16),
        mesh=mesh,
        scratch_types=[
            pltpu.VMEM((BCH, CHUNK), jnp.int32),
            pltpu.VMEM((BCH, CHUNK), jnp.int32),
            pltpu.VMEM((BCH, CHUNK), jnp.int32),
            pltpu.VMEM((BCH, CHUNK), jnp.int32),
            pltpu.VMEM((CHUNK, DA), jnp.bfloat16),
            pltpu.VMEM((CHUNK, DA), jnp.bfloat16),
            pltpu.VMEM_SHARED((ACC_ROWS, DA), jnp.bfloat16),
            pltpu.VMEM_SHARED((ACC_ROWS, DA), jnp.bfloat16),
            pltpu.SemaphoreType.DMA,
            pltpu.SemaphoreType.DMA,
            pltpu.SemaphoreType.DMA,
            pltpu.SemaphoreType.DMA,
            pltpu.SemaphoreType.DMA,
            pltpu.SemaphoreType.DMA,
        ],
        compiler_params=pltpu.CompilerParams(use_tc_tiling_on_sc=False),
    )


R_BLK = 2000


def _proj_body(x_ref, w_ref, b_ref, o_ref):
    y = jnp.maximum(
        jnp.dot(x_ref[0], w_ref[0], preferred_element_type=jnp.float32)
        + b_ref[0], 0.0)
    o_ref[0, :, :D] = y.astype(jnp.bfloat16)
    o_ref[0, :, D:DA] = jnp.ones((y.shape[0], DA - D), jnp.bfloat16)


def _proj(xs, wp, bp):
    return pl.pallas_call(
        _proj_body,
        grid=(2, N // R_BLK),
        in_specs=[
            pl.BlockSpec((1, R_BLK, D), lambda t, r: (t, r, 0)),
            pl.BlockSpec((1, D, D), lambda t, r: (t, 0, 0)),
            pl.BlockSpec((1, 1, D), lambda t, r: (t, 0, 0)),
        ],
        out_specs=pl.BlockSpec((1, R_BLK, DA), lambda t, r: (t, r, 0)),
        out_shape=jax.ShapeDtypeStruct((2, N, DA), jnp.bfloat16),
    )(xs, wp, bp)


def _sage_block(agg_ref, h_ref, wl_ref, wr_ref, b_ref):
    x = agg_ref[0].astype(jnp.float32)
    inv = 1.0 / jnp.maximum(x[:, D:D + 1], 1.0)
    mean = x[:, :D] * inv
    h = h_ref[0][:, :D].astype(jnp.float32)
    z = (jnp.dot(mean, wl_ref[0], preferred_element_type=jnp.float32)
         + jnp.dot(h, wr_ref[0], preferred_element_type=jnp.float32)
         + b_ref[0])
    return jnp.maximum(z, 0.0)


def _store_aug(o_ref, y):
    o_ref[0, :, :D] = y.astype(jnp.bfloat16)
    o_ref[0, :, D:DA] = jnp.ones((y.shape[0], DA - D), jnp.bfloat16)


def _combine_body(agg_ref, h_ref, wl_ref, wr_ref, b_ref, o_ref):
    _store_aug(o_ref, _sage_block(agg_ref, h_ref, wl_ref, wr_ref, b_ref))


_SAGE_SPECS = [
    pl.BlockSpec((1, R_BLK, DA), lambda t, r: (1 - t, r, 0)),
    pl.BlockSpec((1, R_BLK, DA), lambda t, r: (t, r, 0)),
    pl.BlockSpec((1, D, D), lambda t, r: (t, 0, 0)),
    pl.BlockSpec((1, D, D), lambda t, r: (t, 0, 0)),
    pl.BlockSpec((1, 1, D), lambda t, r: (t, 0, 0)),
]


def _combine(agg, h, wl, wr, b):
    return pl.pallas_call(
        _combine_body,
        grid=(2, N // R_BLK),
        in_specs=_SAGE_SPECS,
        out_specs=pl.BlockSpec((1, R_BLK, DA), lambda t, r: (t, r, 0)),
        out_shape=jax.ShapeDtypeStruct((2, N, DA), jnp.bfloat16),
    )(agg, h, wl, wr, b)


def _combine_final_body(agg_ref, h_ref, wl_ref, wr_ref, b_ref, w1_ref,
                        b1_ref, w2_ref, b2_ref, out_ref, cs_ref):
    t = pl.program_id(0)
    r = pl.program_id(1)
    y = _sage_block(agg_ref, h_ref, wl_ref, wr_ref, b_ref)

    @pl.when((t == 0) & (r == 0))
    def _init():
        cs_ref[...] = jnp.zeros_like(cs_ref)

    cs_ref[...] += jnp.sum(y, axis=0, keepdims=True)

    @pl.when((t == 1) & (r == pl.num_programs(1) - 1))
    def _readout():
        ge = cs_ref[...] * (0.5 / N)
        z = jnp.maximum(
            jnp.dot(ge, w1_ref[...], preferred_element_type=jnp.float32)
            + b1_ref[...], 0.0)
        out_ref[...] = (jnp.dot(z, w2_ref[...],
                                preferred_element_type=jnp.float32)
                        + b2_ref[...])


def _combine_final(agg, h, wl, wr, b, w1, b1, w2, b2):
    C = w2.shape[1]
    return pl.pallas_call(
        _combine_final_body,
        grid=(2, N // R_BLK),
        in_specs=_SAGE_SPECS + [
            pl.BlockSpec(w1.shape, lambda t, r: (0, 0)),
            pl.BlockSpec(b1.shape, lambda t, r: (0, 0)),
            pl.BlockSpec(w2.shape, lambda t, r: (0, 0)),
            pl.BlockSpec(b2.shape, lambda t, r: (0, 0)),
        ],
        out_specs=pl.BlockSpec((1, C), lambda t, r: (0, 0)),
        out_shape=jax.ShapeDtypeStruct((1, C), jnp.float32),
        scratch_shapes=[pltpu.VMEM((1, D), jnp.float32)],
    )(agg, h, wl, wr, b, w1, b1, w2, b2)


def _pad_idx(idx, fill, offset):
    a = idx.astype(jnp.int32).reshape(NS, E_TILE) + offset
    pad = jnp.full((NS, E_PAD - E_TILE), fill, jnp.int32)
    return jnp.concatenate([a, pad], axis=1).reshape(NS, NB, BCH, CHUNK)


def kernel(x_user, x_item, edge_index_ui, edge_index_iu, params):
    p = params

    xs = jnp.stack([x_user, x_item])
    wp = jnp.stack([p["Wp_user"], p["Wp_item"]])
    bp = jnp.stack([p["bp_user"], p["bp_item"]])[:, None, :]

    inv_eps = 1.0 / jnp.sqrt(jnp.float32(1.0 + 1e-5))
    wl, wr, bb = [], [], []
    for l in range(2):
        s_u = p[f"g_user_{l}"] * inv_eps
        s_i = p[f"g_item_{l}"] * inv_eps
        wl.append(jnp.stack([p[f"Wl_iu_{l}"] * s_u, p[f"Wl_ui_{l}"] * s_i]))
        wr.append(jnp.stack([p[f"Wr_iu_{l}"] * s_u, p[f"Wr_ui_{l}"] * s_i]))
        bb.append(jnp.stack([
            p[f"bl_iu_{l}"] * s_u + p[f"be_user_{l}"],
            p[f"bl_ui_{l}"] * s_i + p[f"be_item_{l}"],
        ])[:, None, :])

    src_idx = jnp.stack([
        _pad_idx(edge_index_ui[0], 0, 0),
        _pad_idx(edge_index_iu[0], 0, 0),
    ])
    dst_idx = jnp.stack([
        _pad_idx(edge_index_ui[1], N, 0),
        _pad_idx(edge_index_iu[1], N, 0),
    ])

    agg_fn = _make_agg()

    h = _proj(xs, wp, bp)
    agg = agg_fn(h, src_idx, dst_idx)
    h = _combine(agg, h, wl[0], wr[0], bb[0])
    agg = agg_fn(h, src_idx, dst_idx)
    out = _combine_final(agg, h, wl[1], wr[1], bb[1],
                         p["W1"], p["b1"][None, :], p["W2"], p["b2"][None, :])
    return (out, jnp.zeros((), jnp.float32))


# speedup vs baseline: 1.1669x; 1.1398x over previous
import functools

import jax
import jax.numpy as jnp
from jax import lax
from jax.experimental import pallas as pl
from jax.experimental.pallas import tpu as pltpu
from jax.experimental.pallas import tpu_sc as plsc

N = 10000
E = 320000
D = 128
DA = 160
NS = 16
E_TILE = E // NS
CHUNK = 128
BCH = 8
NB = 20
NCH = NB * BCH
E_PAD = NCH * CHUNK
ACC_ROWS = 10240
ROWS_PER_TILE = ACC_ROWS // NS


def _agg_body(da, table, srcs, dsts, out, ixs0, ixs1, ixd0, ixd1, rows0,
              rows1, tb, acc, isem0, isem1, gsem0, gsem1, ssem0, ssem1):
    c = lax.axis_index("c")
    s = lax.axis_index("s")
    ixs = (ixs0, ixs1)
    ixd = (ixd0, ixd1)
    rows = (rows0, rows1)
    isem = (isem0, isem1)
    gsem = (gsem0, gsem1)
    ssem = (ssem0, ssem1)
    base = s * ROWS_PER_TILE

    def stage_block(off, nr):
        pltpu.sync_copy(table.at[c, pl.ds(off, nr)], tb.at[pl.ds(off, nr)])

    @pl.when(s < NS - 1)
    def _stage_full():
        stage_block(base, ROWS_PER_TILE)

    @pl.when(s == NS - 1)
    def _stage_last():
        stage_block(base, N - (NS - 1) * ROWS_PER_TILE)

    zero32 = jnp.zeros((32,), jnp.bfloat16)

    def zrow(i, _):
        def zcol(j, _):
            rows1[i, pl.ds(j * 32, 32)] = zero32
            return 0
        return lax.fori_loop(0, da // 32, zcol, 0)

    lax.fori_loop(0, CHUNK, zrow, 0)
    nfull = ROWS_PER_TILE // CHUNK
    for k in range(nfull):
        pltpu.sync_copy(rows1, acc.at[pl.ds(base + k * CHUNK, CHUNK)])
    rem = ROWS_PER_TILE - nfull * CHUNK
    if rem:
        pltpu.sync_copy(rows1.at[pl.ds(0, rem)],
                        acc.at[pl.ds(base + nfull * CHUNK, rem)])

    plsc.subcore_barrier()

    def load_batch(m, a):
        pltpu.async_copy(srcs.at[c, s, m], ixs[a], isem[a])
        pltpu.async_copy(dsts.at[c, s, m], ixd[a], isem[a])

    def wait_batch(a):
        pltpu.make_async_copy(srcs.at[c, s, 0], ixs[a], isem[a]).wait()
        pltpu.make_async_copy(dsts.at[c, s, 0], ixd[a], isem[a]).wait()

    def gather(a, k, b):
        pltpu.async_copy(tb.at[ixs[a].at[k]], rows[b], gsem[b])

    def wait_gather(a, k, b):
        pltpu.make_async_copy(tb.at[ixs[a].at[k]], rows[b], gsem[b]).wait()

    load_batch(0, 0)
    load_batch(1, 1)
    wait_batch(0)
    gather(0, 0, 0)
    gather(0, 1, 1)

    def superstep(t, _):
        for a in (0, 1):
            m = 2 * t + a
            for k in range(BCH):
                b = k % 2
                wait_gather(a, k, b)
                pltpu.async_copy(rows[b], acc.at[ixd[a].at[k]], ssem[b],
                                 add=True)
                pltpu.make_async_copy(rows[b], acc.at[ixd[a].at[k]],
                                      ssem[b]).wait()
                if k < BCH - 2:
                    gather(a, k + 2, b)
                else:
                    if k == BCH - 2:
                        @pl.when(m < NB - 1)
                        def _wait_next_idx():
                            wait_batch(1 - a)

                    @pl.when(m < NB - 1)
                    def _cross_batch_gather():
                        gather(1 - a, k + 2 - BCH, b)

            @pl.when(m + 2 < NB)
            def _prefetch_idx():
                load_batch(m + 2, a)
        return 0

    lax.fori_loop(0, NB // 2, superstep, 0)

    plsc.subcore_barrier()

    @pl.when(s < NS - 1)
    def _full_tile():
        pltpu.sync_copy(acc.at[pl.ds(base, ROWS_PER_TILE)],
                        out.at[c, pl.ds(base, ROWS_PER_TILE)])

    @pl.when(s == NS - 1)
    def _last_tile():
        tail = N - (NS - 1) * ROWS_PER_TILE
        pltpu.sync_copy(acc.at[pl.ds(base, tail)],
                        out.at[c, pl.ds(base, tail)])


def _make_agg(da):
    mesh = plsc.VectorSubcoreMesh(core_axis_name="c", subcore_axis_name="s",
                                  num_cores=2, num_subcores=NS)
    return pl.kernel(
        functools.partial(_agg_body, da),
        out_type=jax.ShapeDtypeStruct((2, N, da), jnp.bfloat16),
        mesh=mesh,
        scratch_types=[
            pltpu.VMEM((BCH, CHUNK), jnp.int32),
            pltpu.VMEM((BCH, CHUNK), jnp.int32),
            pltpu.VMEM((BCH, CHUNK), jnp.int32),
            pltpu.VMEM((BCH, CHUNK), jnp.int32),
            pltpu.VMEM((CHUNK, da), jnp.bfloat16),
            pltpu.VMEM((CHUNK, da), jnp.bfloat16),
            pltpu.VMEM_SHARED((ACC_ROWS, da), jnp.bfloat16),
            pltpu.VMEM_SHARED((ACC_ROWS, da), jnp.bfloat16),
            pltpu.SemaphoreType.DMA,
            pltpu.SemaphoreType.DMA,
            pltpu.SemaphoreType.DMA,
            pltpu.SemaphoreType.DMA,
            pltpu.SemaphoreType.DMA,
            pltpu.SemaphoreType.DMA,
        ],
        compiler_params=pltpu.CompilerParams(use_tc_tiling_on_sc=False),
    )


R_BLK = 2000


def _proj_body(x_ref, w_ref, b_ref, o_ref):
    y = jnp.maximum(
        jnp.dot(x_ref[0], w_ref[0], preferred_element_type=jnp.float32)
        + b_ref[0], 0.0)
    o_ref[0, :, :D] = y.astype(jnp.bfloat16)
    o_ref[0, :, D:DA] = jnp.ones((y.shape[0], DA - D), jnp.bfloat16)


def _proj(xs, wp, bp):
    return pl.pallas_call(
        _proj_body,
        grid=(2, N // R_BLK),
        in_specs=[
            pl.BlockSpec((1, R_BLK, D), lambda t, r: (t, r, 0)),
            pl.BlockSpec((1, D, D), lambda t, r: (t, 0, 0)),
            pl.BlockSpec((1, 1, D), lambda t, r: (t, 0, 0)),
        ],
        out_specs=pl.BlockSpec((1, R_BLK, DA), lambda t, r: (t, r, 0)),
        out_shape=jax.ShapeDtypeStruct((2, N, DA), jnp.bfloat16),
    )(xs, wp, bp)


def _sage_block(agg_ref, cnt_ref, h_ref, wl_ref, wr_ref, b_ref):
    x = agg_ref[0].astype(jnp.float32)
    cnt = cnt_ref[0][:, D:D + 1].astype(jnp.float32)
    inv = 1.0 / jnp.maximum(cnt, 1.0)
    mean = x[:, :D] * inv
    h = h_ref[0][:, :D].astype(jnp.float32)
    z = (jnp.dot(mean, wl_ref[0], preferred_element_type=jnp.float32)
         + jnp.dot(h, wr_ref[0], preferred_element_type=jnp.float32)
         + b_ref[0])
    return jnp.maximum(z, 0.0)


def _combine_body(agg_ref, h_ref, wl_ref, wr_ref, b_ref, o_ref):
    y = _sage_block(agg_ref, agg_ref, h_ref, wl_ref, wr_ref, b_ref)
    o_ref[0] = y.astype(jnp.bfloat16)


def _sage_specs(wa, wh):
    return [
        pl.BlockSpec((1, R_BLK, wa), lambda t, r: (1 - t, r, 0)),
        pl.BlockSpec((1, R_BLK, wh), lambda t, r: (t, r, 0)),
        pl.BlockSpec((1, D, D), lambda t, r: (t, 0, 0)),
        pl.BlockSpec((1, D, D), lambda t, r: (t, 0, 0)),
        pl.BlockSpec((1, 1, D), lambda t, r: (t, 0, 0)),
    ]


def _combine(agg, h, wl, wr, b):
    return pl.pallas_call(
        _combine_body,
        grid=(2, N // R_BLK),
        in_specs=_sage_specs(DA, DA),
        out_specs=pl.BlockSpec((1, R_BLK, D), lambda t, r: (t, r, 0)),
        out_shape=jax.ShapeDtypeStruct((2, N, D), jnp.bfloat16),
    )(agg, h, wl, wr, b)


def _combine_final_body(agg_ref, h_ref, wl_ref, wr_ref, b_ref, cnt_ref,
                        w1_ref, b1_ref, w2_ref, b2_ref, out_ref, cs_ref):
    t = pl.program_id(0)
    r = pl.program_id(1)
    y = _sage_block(agg_ref, cnt_ref, h_ref, wl_ref, wr_ref, b_ref)

    @pl.when((t == 0) & (r == 0))
    def _init():
        cs_ref[...] = jnp.zeros_like(cs_ref)

    cs_ref[...] += jnp.sum(y, axis=0, keepdims=True)

    @pl.when((t == 1) & (r == pl.num_programs(1) - 1))
    def _readout():
        ge = cs_ref[...] * (0.5 / N)
        z = jnp.maximum(
            jnp.dot(ge, w1_ref[...], preferred_element_type=jnp.float32)
            + b1_ref[...], 0.0)
        out_ref[...] = (jnp.dot(z, w2_ref[...],
                                preferred_element_type=jnp.float32)
                        + b2_ref[...])


def _combine_final(agg, h, wl, wr, b, cnt, w1, b1, w2, b2):
    C = w2.shape[1]
    return pl.pallas_call(
        _combine_final_body,
        grid=(2, N // R_BLK),
        in_specs=_sage_specs(D, D) + [
            pl.BlockSpec((1, R_BLK, DA), lambda t, r: (1 - t, r, 0)),
            pl.BlockSpec(w1.shape, lambda t, r: (0, 0)),
            pl.BlockSpec(b1.shape, lambda t, r: (0, 0)),
            pl.BlockSpec(w2.shape, lambda t, r: (0, 0)),
            pl.BlockSpec(b2.shape, lambda t, r: (0, 0)),
        ],
        out_specs=pl.BlockSpec((1, C), lambda t, r: (0, 0)),
        out_shape=jax.ShapeDtypeStruct((1, C), jnp.float32),
        scratch_shapes=[pltpu.VMEM((1, D), jnp.float32)],
    )(agg, h, wl, wr, b, cnt, w1, b1, w2, b2)


def _pad_idx(idx, fill, offset):
    a = idx.astype(jnp.int32).reshape(NS, E_TILE) + offset
    pad = jnp.full((NS, E_PAD - E_TILE), fill, jnp.int32)
    return jnp.concatenate([a, pad], axis=1).reshape(NS, NB, BCH, CHUNK)


def kernel(x_user, x_item, edge_index_ui, edge_index_iu, params):
    p = params

    xs = jnp.stack([x_user, x_item])
    wp = jnp.stack([p["Wp_user"], p["Wp_item"]])
    bp = jnp.stack([p["bp_user"], p["bp_item"]])[:, None, :]

    inv_eps = 1.0 / jnp.sqrt(jnp.float32(1.0 + 1e-5))
    wl, wr, bb = [], [], []
    for l in range(2):
        s_u = p[f"g_user_{l}"] * inv_eps
        s_i = p[f"g_item_{l}"] * inv_eps
        wl.append(jnp.stack([p[f"Wl_iu_{l}"] * s_u, p[f"Wl_ui_{l}"] * s_i]))
        wr.append(jnp.stack([p[f"Wr_iu_{l}"] * s_u, p[f"Wr_ui_{l}"] * s_i]))
        bb.append(jnp.stack([
            p[f"bl_iu_{l}"] * s_u + p[f"be_user_{l}"],
            p[f"bl_ui_{l}"] * s_i + p[f"be_item_{l}"],
        ])[:, None, :])

    src_idx = jnp.stack([
        _pad_idx(edge_index_ui[0], 0, 0),
        _pad_idx(edge_index_iu[0], 0, 0),
    ])
    dst_idx = jnp.stack([
        _pad_idx(edge_index_ui[1], N, 0),
        _pad_idx(edge_index_iu[1], N, 0),
    ])

    h = _proj(xs, wp, bp)
    agg0 = _make_agg(DA)(h, src_idx, dst_idx)
    h = _combine(agg0, h, wl[0], wr[0], bb[0])
    agg1 = _make_agg(D)(h, src_idx, dst_idx)
    out = _combine_final(agg1, h, wl[1], wr[1], bb[1], agg0,
                         p["W1"], p["b1"][None, :], p["W2"], p["b2"][None, :])
    return (out, jnp.zeros((), jnp.float32))
